# Initial kernel scaffold; baseline (speedup 1.0000x reference)
import jax
import jax.numpy as jnp
from jax.experimental import pallas as pl


def kernel(points, Wqkv, bqkv, g1, b1, Wproj, bproj):
    raise NotImplementedError("write your pallas kernel here")


# speedup vs baseline: 5.1759x; 5.1759x over previous
import functools

import jax
import jax.numpy as jnp
from jax.experimental import pallas as pl
from jax.experimental.pallas import tpu as pltpu

H = 16
TOPK = 16


def _ln_qkv_body(x_ref, w_ref, b_ref, g_ref, bt_ref, q_ref, kv_ref, *, D):
    x = x_ref[...]
    mu = jnp.mean(x, axis=1, keepdims=True)
    xc = x - mu
    var = jnp.mean(xc * xc, axis=1, keepdims=True)
    xn = xc / jnp.sqrt(var + 1e-5) * g_ref[...] + bt_ref[...]
    y = jnp.dot(xn, w_ref[...], preferred_element_type=jnp.float32) + b_ref[...]
    q_ref[...] = y[:, :D]
    kv_ref[...] = y[:, D:]


def _attn_body(q_ref, kv_ref, vres_ref, wp_ref, bp_ref, msg_ref, top_ref,
               *, N, D, C, temp):
    q = q_ref[...]
    s_sum = None
    o_parts = []
    for h in range(H):
        qh = q[:, h * C:(h + 1) * C]
        kh = kv_ref[:, h * C:(h + 1) * C]
        s = jax.lax.dot_general(qh, kh, (((1,), (1,)), ((), ())),
                                preferred_element_type=jnp.float32)
        s = s * temp
        m = jnp.max(s, axis=1, keepdims=True)
        e = jnp.exp(s - m)
        z = jnp.sum(e, axis=1, keepdims=True)
        p = e / z
        s_sum = p if s_sum is None else s_sum + p
        vh = kv_ref[:, D + h * C:D + (h + 1) * C]
        o_parts.append(jnp.dot(p, vh, preferred_element_type=jnp.float32))
    o = jnp.concatenate(o_parts, axis=1)
    msg = (jnp.dot(o, wp_ref[...], preferred_element_type=jnp.float32)
           + bp_ref[...] + vres_ref[...])
    msg_ref[...] = msg
    iota = jax.lax.broadcasted_iota(jnp.int32, s_sum.shape, 1)
    cols = []
    s_work = s_sum
    for _ in range(TOPK):
        m = jnp.max(s_work, axis=1, keepdims=True)
        idx = jnp.min(jnp.where(s_work == m, iota, N), axis=1, keepdims=True)
        cols.append(idx)
        s_work = jnp.where(iota == idx, -jnp.inf, s_work)
    top_ref[...] = jnp.concatenate(cols, axis=1)


def kernel(points, Wqkv, bqkv, g1, b1, Wproj, bproj):
    N, D = points.shape
    C = D // H
    BQ = min(256, N)
    grid = N // BQ
    temp = 1.0 / (C ** 0.5)

    q, kv = pl.pallas_call(
        functools.partial(_ln_qkv_body, D=D),
        grid=(grid,),
        in_specs=[
            pl.BlockSpec((BQ, D), lambda i: (i, 0)),
            pl.BlockSpec((D, 3 * D), lambda i: (0, 0)),
            pl.BlockSpec((1, 3 * D), lambda i: (0, 0)),
            pl.BlockSpec((1, D), lambda i: (0, 0)),
            pl.BlockSpec((1, D), lambda i: (0, 0)),
        ],
        out_specs=[
            pl.BlockSpec((BQ, D), lambda i: (i, 0)),
            pl.BlockSpec((BQ, 2 * D), lambda i: (i, 0)),
        ],
        out_shape=[
            jax.ShapeDtypeStruct((N, D), jnp.float32),
            jax.ShapeDtypeStruct((N, 2 * D), jnp.float32),
        ],
        compiler_params=pltpu.CompilerParams(
            dimension_semantics=("arbitrary",),
            vmem_limit_bytes=128 * 1024 * 1024,
        ),
    )(points, Wqkv, bqkv.reshape(1, -1), g1.reshape(1, -1), b1.reshape(1, -1))

    msg, top = pl.pallas_call(
        functools.partial(_attn_body, N=N, D=D, C=C, temp=temp),
        grid=(grid,),
        in_specs=[
            pl.BlockSpec((BQ, D), lambda i: (i, 0)),
            pl.BlockSpec((N, 2 * D), lambda i: (0, 0)),
            pl.BlockSpec((BQ, D), lambda i: (i, 1)),
            pl.BlockSpec((D, D), lambda i: (0, 0)),
            pl.BlockSpec((1, D), lambda i: (0, 0)),
        ],
        out_specs=[
            pl.BlockSpec((BQ, D), lambda i: (i, 0)),
            pl.BlockSpec((BQ, TOPK), lambda i: (i, 0)),
        ],
        out_shape=[
            jax.ShapeDtypeStruct((N, D), jnp.float32),
            jax.ShapeDtypeStruct((N, TOPK), jnp.int32),
        ],
        compiler_params=pltpu.CompilerParams(
            dimension_semantics=("arbitrary",),
            vmem_limit_bytes=128 * 1024 * 1024,
        ),
    )(q, kv, kv, Wproj, bproj.reshape(1, -1))
    return msg, top
